# baseline (speedup 1.0000x reference)
import functools

import jax
import jax.numpy as jnp
import numpy as np
from jax import lax
from jax.experimental import pallas as pl
from jax.experimental.pallas import tpu as pltpu
from jax.experimental.pallas import tpu_sc as plsc

B = 2
N = 32768
C = 512
DHW = 32
NV = DHW ** 3

NW = 32
PPW = N // NW
P = 16
NBLK = PPW // P

_ROWS = (256, 128)

VB = 2048


def _prep_body(x_ref, y_ref, z_ref, idx_ref, w_ref):
    x = x_ref[...]
    y = y_ref[...]
    z = z_ref[...]
    scale = 0.5 * (DHW - 1)

    def split(v):
        iv = (v + 1.0) * scale
        v0f = jnp.floor(iv)
        f1 = iv - v0f
        f0 = 1.0 - f1
        v0 = jnp.clip(v0f.astype(jnp.int32), 0, DHW - 1)
        v1 = jnp.clip(v0 + 1, 0, DHW - 1)
        return (v0, v1), (f0, f1)

    (x0, x1), (fx0, fx1) = split(x)
    (y0, y1), (fy0, fy1) = split(y)
    (z0, z1), (fz0, fz1) = split(z)

    xs = (x0, x1)
    ys = (y0, y1)
    zs = (z0, z1)
    fxs = (fx0, fx1)
    fys = (fy0, fy1)
    fzs = (fz0, fz1)
    for zb in range(2):
        for yb in range(2):
            for xb in range(2):
                k = zb * 4 + yb * 2 + xb
                idx_ref[k] = zs[zb] * (DHW * DHW) + ys[yb] * DHW + xs[xb]
                w_ref[k] = fzs[zb] * fys[yb] * fxs[xb]


def _prep(x, y, z):
    return pl.pallas_call(
        _prep_body,
        out_shape=(
            jax.ShapeDtypeStruct((8,) + _ROWS, jnp.int32),
            jax.ShapeDtypeStruct((8,) + _ROWS, jnp.float32),
        ),
    )(x, y, z)


def _pack_body(x_ref, t_ref):
    x = x_ref[...]
    lo = lax.bitcast_convert_type(x[:, : C // 2], jnp.uint16).astype(jnp.int32)
    hi = lax.bitcast_convert_type(x[:, C // 2 :], jnp.uint16).astype(jnp.int32)
    t_ref[...] = lo | (hi << 16)


def _pack(table_bf16):
    return pl.pallas_call(
        _pack_body,
        grid=(NV // VB,),
        in_specs=[pl.BlockSpec((VB, C), lambda i: (i, 0))],
        out_specs=pl.BlockSpec((VB, C // 2), lambda i: (i, 0)),
        out_shape=jax.ShapeDtypeStruct((NV, C // 2), jnp.int32),
    )(table_bf16)


def _sc_body(idx_hbm, w_hbm, table_hbm, out_hbm, idx_v, w_v, rows_v, out_v,
             gsem, osem):
    wid = lax.axis_index("s") * 2 + lax.axis_index("c")
    base0 = wid * PPW
    pltpu.sync_copy(idx_hbm.at[wid], idx_v)
    pltpu.sync_copy(w_hbm.at[wid], w_v)

    def gather_desc(t):
        par = t % 2
        return pltpu.make_async_copy(
            table_hbm.at[idx_v.at[t]], rows_v.at[par], gsem.at[par])

    def out_desc(t):
        par = t % 2
        dst = out_hbm.at[pl.ds(base0 + t * P, P)]
        return pltpu.make_async_copy(out_v.at[par], dst, osem.at[par])

    gather_desc(0).start()

    def job(t, carry):
        par = t % 2

        @pl.when(t + 1 < NBLK)
        def _():
            gather_desc(t + 1).start()

        gather_desc(t).wait()

        @pl.when(t >= 2)
        def _():
            out_desc(t - 2).wait()

        wrows = [w_v[t, pl.ds(k * P, P)] for k in range(8)]

        @plsc.parallel_loop(0, P, unroll=2)
        def pbody(p):
            ps = jnp.full((16,), p, jnp.int32)
            wv = [wr.at[ps].get(mode="promise_in_bounds") for wr in wrows]
            for t16 in range(C // 32):
                s = pl.ds(t16 * 16, 16)
                acc_a = None
                acc_b = None
                for k in range(8):
                    bits = rows_v[par, k * P + p, s]
                    a = plsc.bitcast(bits << 16, jnp.float32)
                    b = plsc.bitcast(bits & jnp.int32(-65536), jnp.float32)
                    if acc_a is None:
                        acc_a = a * wv[k]
                        acc_b = b * wv[k]
                    else:
                        acc_a = acc_a + a * wv[k]
                        acc_b = acc_b + b * wv[k]
                out_v[par, p, pl.ds(t16 * 16, 16)] = acc_a
                out_v[par, p, pl.ds(C // 2 + t16 * 16, 16)] = acc_b

        out_desc(t).start()
        return carry

    lax.fori_loop(0, NBLK, job, 0)
    out_desc(NBLK - 2).wait()
    out_desc(NBLK - 1).wait()


_sc_gather = functools.partial(
    pl.kernel,
    out_type=jax.ShapeDtypeStruct((N, C), jnp.float32),
    mesh=plsc.VectorSubcoreMesh(core_axis_name="c", subcore_axis_name="s"),
    compiler_params=pltpu.CompilerParams(needs_layout_passes=False),
    scratch_types=[
        pltpu.VMEM((NBLK, 8 * P), jnp.int32),
        pltpu.VMEM((NBLK, 8 * P), jnp.float32),
        pltpu.VMEM((2, 8 * P, C // 2), jnp.int32),
        pltpu.VMEM((2, P, C), jnp.float32),
        pltpu.SemaphoreType.DMA((2,)),
        pltpu.SemaphoreType.DMA((2,)),
    ],
)(_sc_body)


def kernel(voxel_features, vertices):
    outs = []
    for b in range(B):
        table_bf16 = (jnp.transpose(voxel_features[b], (1, 2, 3, 0))
                      .reshape(NV, C).astype(jnp.bfloat16))
        table = _pack(table_bf16)
        x = vertices[b, :, 0].reshape(_ROWS)
        y = vertices[b, :, 1].reshape(_ROWS)
        z = vertices[b, :, 2].reshape(_ROWS)
        idx8, w8 = _prep(x, y, z)
        idx_t = (idx8.reshape(8, NW, NBLK, P)
                 .transpose(1, 2, 0, 3).reshape(NW, NBLK, 8 * P))
        w_t = (w8.reshape(8, NW, NBLK, P)
               .transpose(1, 2, 0, 3).reshape(NW, NBLK, 8 * P))
        outs.append(_sc_gather(idx_t, w_t, table))
    return jnp.stack(outs)


# speedup vs baseline: 1.3004x; 1.3004x over previous
import functools

import jax
import jax.numpy as jnp
import numpy as np
from jax import lax
from jax.experimental import pallas as pl
from jax.experimental.pallas import tpu as pltpu
from jax.experimental.pallas import tpu_sc as plsc

B = 2
N = 32768
C = 512
DHW = 32
BN = B * N

NW = 32
PPW = BN // NW
P = 16
NBLK = PPW // P

_ROWS = (512, 128)

VB = 2048
NV = DHW ** 3


def _prep_body(x_ref, y_ref, z_ref, idx_ref, w_ref):
    x = x_ref[...]
    y = y_ref[...]
    z = z_ref[...]
    scale = 0.5 * (DHW - 1)

    def split(v):
        iv = (v + 1.0) * scale
        v0f = jnp.floor(iv)
        f1 = iv - v0f
        f0 = 1.0 - f1
        v0 = jnp.clip(v0f.astype(jnp.int32), 0, DHW - 1)
        v1 = jnp.clip(v0 + 1, 0, DHW - 1)
        return (v0, v1), (f0, f1)

    (x0, x1), (fx0, fx1) = split(x)
    (y0, y1), (fy0, fy1) = split(y)
    (z0, z1), (fz0, fz1) = split(z)

    r = lax.broadcasted_iota(jnp.int32, _ROWS, 0)
    boff = jnp.where(r >= _ROWS[0] // B, N, 0)

    xs = (x0, x1)
    ys = (y0, y1)
    zs = (z0, z1)
    fxs = (fx0, fx1)
    fys = (fy0, fy1)
    fzs = (fz0, fz1)
    for zb in range(2):
        for yb in range(2):
            for xb in range(2):
                k = zb * 4 + yb * 2 + xb
                idx_ref[k] = boff + zs[zb] * (DHW * DHW) + ys[yb] * DHW + xs[xb]
                w_ref[k] = fzs[zb] * fys[yb] * fxs[xb]


def _prep(x, y, z):
    return pl.pallas_call(
        _prep_body,
        out_shape=(
            jax.ShapeDtypeStruct((8,) + _ROWS, jnp.int32),
            jax.ShapeDtypeStruct((8,) + _ROWS, jnp.float32),
        ),
    )(x, y, z)


def _pack_body(x_ref, t_ref):
    x = x_ref[...]
    lo = lax.bitcast_convert_type(x[:, : C // 2], jnp.uint16).astype(jnp.int32)
    hi = lax.bitcast_convert_type(x[:, C // 2 :], jnp.uint16).astype(jnp.int32)
    t_ref[...] = lo | (hi << 16)


def _pack(table_bf16):
    return pl.pallas_call(
        _pack_body,
        grid=(BN // VB,),
        in_specs=[pl.BlockSpec((VB, C), lambda i: (i, 0))],
        out_specs=pl.BlockSpec((VB, C // 2), lambda i: (i, 0)),
        out_shape=jax.ShapeDtypeStruct((BN, C // 2), jnp.int32),
    )(table_bf16)


def _sc_body(idx_hbm, w_hbm, table_hbm, out_hbm, idx_v, w_v, rows_v, out_v,
             gsem, osem):
    wid = lax.axis_index("s") * 2 + lax.axis_index("c")
    base0 = wid * PPW
    pltpu.sync_copy(idx_hbm.at[wid], idx_v)
    pltpu.sync_copy(w_hbm.at[wid], w_v)

    def gather_desc(t):
        par = t % 2
        return pltpu.make_async_copy(
            table_hbm.at[idx_v.at[t]], rows_v.at[par], gsem.at[par])

    def out_desc(t):
        par = t % 2
        dst = out_hbm.at[pl.ds(base0 + t * P, P)]
        return pltpu.make_async_copy(out_v.at[par], dst, osem.at[par])

    gather_desc(0).start()

    def job(t, carry):
        par = t % 2

        @pl.when(t + 1 < NBLK)
        def _():
            gather_desc(t + 1).start()

        gather_desc(t).wait()

        @pl.when(t >= 2)
        def _():
            out_desc(t - 2).wait()

        wrows = [w_v[t, pl.ds(k * P, P)] for k in range(8)]

        @plsc.parallel_loop(0, P, unroll=2)
        def pbody(p):
            ps = jnp.full((16,), p, jnp.int32)
            wv = [wr.at[ps].get(mode="promise_in_bounds") for wr in wrows]
            for t16 in range(C // 32):
                s = pl.ds(t16 * 16, 16)
                acc_a = None
                acc_b = None
                for k in range(8):
                    bits = rows_v[par, k * P + p, s]
                    a = plsc.bitcast(bits << 16, jnp.float32)
                    b = plsc.bitcast(bits, jnp.float32)
                    if acc_a is None:
                        acc_a = a * wv[k]
                        acc_b = b * wv[k]
                    else:
                        acc_a = acc_a + a * wv[k]
                        acc_b = acc_b + b * wv[k]
                out_v[par, p, pl.ds(t16 * 16, 16)] = acc_a
                out_v[par, p, pl.ds(C // 2 + t16 * 16, 16)] = acc_b

        out_desc(t).start()
        return carry

    lax.fori_loop(0, NBLK, job, 0)
    out_desc(NBLK - 2).wait()
    out_desc(NBLK - 1).wait()


_sc_gather = functools.partial(
    pl.kernel,
    out_type=jax.ShapeDtypeStruct((BN, C), jnp.float32),
    mesh=plsc.VectorSubcoreMesh(core_axis_name="c", subcore_axis_name="s"),
    compiler_params=pltpu.CompilerParams(needs_layout_passes=False),
    scratch_types=[
        pltpu.VMEM((NBLK, 8 * P), jnp.int32),
        pltpu.VMEM((NBLK, 8 * P), jnp.float32),
        pltpu.VMEM((2, 8 * P, C // 2), jnp.int32),
        pltpu.VMEM((2, P, C), jnp.float32),
        pltpu.SemaphoreType.DMA((2,)),
        pltpu.SemaphoreType.DMA((2,)),
    ],
)(_sc_body)


def kernel(voxel_features, vertices):
    table_bf16 = (jnp.transpose(voxel_features, (0, 2, 3, 4, 1))
                  .reshape(BN, C).astype(jnp.bfloat16))
    table = _pack(table_bf16)
    v = vertices.reshape(BN, 3)
    x = v[:, 0].reshape(_ROWS)
    y = v[:, 1].reshape(_ROWS)
    z = v[:, 2].reshape(_ROWS)
    idx8, w8 = _prep(x, y, z)
    idx_t = idx8.reshape(8, NW, NBLK, P).transpose(1, 2, 0, 3).reshape(NW, NBLK, 8 * P)
    w_t = w8.reshape(8, NW, NBLK, P).transpose(1, 2, 0, 3).reshape(NW, NBLK, 8 * P)
    out = _sc_gather(idx_t, w_t, table)
    return out.reshape(B, N, C)
